# Initial kernel scaffold; baseline (speedup 1.0000x reference)
import jax
import jax.numpy as jnp
from jax.experimental import pallas as pl


def kernel(x, edge_index, edge_attr, residue_type, batch, W0, Wr0, b0, rw0, W1, Wr1, b1, rw1, W2, Wr2, b2, rw2, mW1, mb1, mW2, mb2):
    raise NotImplementedError("write your pallas kernel here")


# speedup vs baseline: 1.7172x; 1.7172x over previous
import functools

import jax
import jax.numpy as jnp
from jax import lax
from jax.experimental import pallas as pl
from jax.experimental.pallas import tpu as pltpu
from jax.experimental.pallas import tpu_sc as plsc

N = 10000
E = 160000
D = 256
NG = 8
NUM_EDGE_TYPES = 400
RW_DIM = 448
NPAD = 10240
NODES_PER_TILE = 320
NTILES = 32
BLK = 4096
EPAD = 172032
NBLK = EPAD // BLK
GATHER_CHUNK = 64
PAD_DST = 1 << 29

_f32 = jnp.float32
_i32 = jnp.int32


def _sc_mesh():
    return plsc.VectorSubcoreMesh(core_axis_name="c", subcore_axis_name="s")


def _wid():
    return lax.axis_index("s") * 2 + lax.axis_index("c")


def _deg_kernel(dst_hbm, et_hbm, relw_hbm, zeros_hbm, part_hbm,
                dst_v, et_v, relw_v, acc_v):
    wid = _wid()
    epw = EPAD // NTILES
    base = wid * epw
    pltpu.sync_copy(dst_hbm.at[pl.ds(base, epw)], dst_v)
    pltpu.sync_copy(et_hbm.at[pl.ds(base, epw)], et_v)
    pltpu.sync_copy(relw_hbm, relw_v)
    pltpu.sync_copy(zeros_hbm, acc_v)
    col = lax.iota(_i32, 16)
    ones16 = jnp.ones((16,), _f32)

    def body(v, _):
        d16 = plsc.load_gather(dst_v, [v * 16 + col])
        e16 = plsc.load_gather(et_v, [v * 16 + col])
        m = d16 < PAD_DST
        dc = jnp.where(m, d16, 0)
        for l in range(3):
            wl = plsc.load_gather(relw_v, [jnp.full((16,), l, _i32), e16])
            plsc.addupdate_scatter(acc_v, [jnp.full((16,), l, _i32), dc],
                                   wl, mask=m)
        plsc.addupdate_scatter(acc_v, [jnp.full((16,), 3, _i32), dc],
                               ones16, mask=m)
        return 0

    lax.fori_loop(0, epw // 16, body, 0)
    pltpu.sync_copy(acc_v, part_hbm.at[wid])


def _run_deg(dst, et, relw_all, zeros4):
    k = pl.kernel(
        _deg_kernel,
        out_type=jax.ShapeDtypeStruct((NTILES, 4, NPAD), _f32),
        mesh=_sc_mesh(),
        scratch_types=[
            pltpu.VMEM((EPAD // NTILES,), _i32),
            pltpu.VMEM((EPAD // NTILES,), _i32),
            pltpu.VMEM((3, RW_DIM), _f32),
            pltpu.VMEM((4, NPAD), _f32),
        ],
        compiler_params=pltpu.CompilerParams(needs_layout_passes=False),
    )
    return k(dst, et, relw_all, zeros4)


def _reduce_kernel(part_ref, out_ref):
    d = jnp.sum(part_ref[...], axis=0)
    row = lax.broadcasted_iota(_i32, d.shape, 0)
    safe = jnp.maximum(d, 1.0)
    out_ref[...] = jnp.where(row < 3, 1.0 / d, 1.0 / safe)


def _run_reduce(part):
    return pl.pallas_call(
        _reduce_kernel,
        out_shape=jax.ShapeDtypeStruct((4, NPAD), _f32),
    )(part)


def _msg_kernel(xw_hbm, dst_hbm, et_hbm, src_hbm, relw_hbm, inv_hbm,
                invc_hbm, zeros_hbm, out_hbm,
                s_dst, s_et, s_src, c_src, c_dstl, c_ew,
                relw_v, inv_v, invc_v, rows_v, agg_v, sem):
    wid = _wid()
    lo = wid * NODES_PER_TILE
    pltpu.sync_copy(relw_hbm, relw_v)
    pltpu.sync_copy(inv_hbm.at[pl.ds(lo, NODES_PER_TILE)], inv_v)
    pltpu.sync_copy(invc_hbm.at[pl.ds(lo, NODES_PER_TILE)], invc_v)
    pltpu.sync_copy(zeros_hbm, agg_v)
    col = lax.iota(_i32, 16)
    zf16 = jnp.zeros((16,), _f32)
    zi16 = jnp.zeros((16,), _i32)

    def block_body(b, _):
        off = b * BLK
        pltpu.sync_copy(dst_hbm.at[pl.ds(off, BLK)], s_dst)
        pltpu.sync_copy(et_hbm.at[pl.ds(off, BLK)], s_et)
        pltpu.sync_copy(src_hbm.at[pl.ds(off, BLK)], s_src)

        def cbody(v, cursor):
            d16 = plsc.load_gather(s_dst, [v * 16 + col])
            m = (d16 >= lo) & (d16 < lo + NODES_PER_TILE)
            e16 = plsc.load_gather(s_et, [v * 16 + col])
            s16 = plsc.load_gather(s_src, [v * 16 + col])
            w16 = plsc.load_gather(relw_v, [e16])
            dl = jnp.where(m, d16 - lo, 0)
            iv = plsc.load_gather(inv_v, [dl])
            ew = w16 * iv
            plsc.store_compressed(c_src.at[pl.ds(cursor, 16)], s16, mask=m)
            plsc.store_compressed(c_dstl.at[pl.ds(cursor, 16)], dl, mask=m)
            plsc.store_compressed(c_ew.at[pl.ds(cursor, 16)], ew, mask=m)
            return cursor + jnp.sum(m.astype(_i32))

        kcnt = lax.fori_loop(0, BLK // 16, cbody, jnp.int32(0))
        for t in range(5):
            plsc.store_scatter(c_ew, [kcnt + t * 16 + col], zf16)
            plsc.store_scatter(c_src, [kcnt + t * 16 + col], zi16)
            plsc.store_scatter(c_dstl, [kcnt + t * 16 + col], zi16)
        nch = (kcnt + GATHER_CHUNK - 1) // GATHER_CHUNK

        def gbody(g, _):
            pltpu.async_copy(
                xw_hbm.at[c_src.at[pl.ds(g * GATHER_CHUNK, GATHER_CHUNK)]],
                rows_v, sem).wait()

            def rbody(k, _):
                idxk = jnp.full((16,), g * GATHER_CHUNK + k, _i32)
                ewk = plsc.load_gather(c_ew, [idxk])
                dk = plsc.load_gather(c_dstl, [idxk])
                kf = jnp.full((16,), k, _i32)
                for j in range(16):
                    val = plsc.load_gather(rows_v, [kf, j * 16 + col]) * ewk
                    plsc.addupdate_scatter(agg_v, [dk, j * 16 + col], val)
                return 0

            lax.fori_loop(0, GATHER_CHUNK, rbody, 0)
            return 0

        lax.fori_loop(0, nch, gbody, 0)
        return 0

    lax.fori_loop(0, NBLK, block_body, 0)

    def fbody(r, _):
        rf = jnp.full((16,), r, _i32)
        ic = plsc.load_gather(invc_v, [rf])
        for j in range(16):
            v = plsc.load_gather(agg_v, [rf, j * 16 + col]) * ic
            plsc.store_scatter(agg_v, [rf, j * 16 + col], v)
        return 0

    lax.fori_loop(0, NODES_PER_TILE, fbody, 0)
    pltpu.sync_copy(agg_v, out_hbm.at[pl.ds(lo, NODES_PER_TILE)])


def _run_msg(xw, dst, et, src, relw_l, inv_l, invc, zeros_a):
    k = pl.kernel(
        _msg_kernel,
        out_type=jax.ShapeDtypeStruct((NPAD, D), _f32),
        mesh=_sc_mesh(),
        scratch_types=[
            pltpu.VMEM((BLK,), _i32),
            pltpu.VMEM((BLK,), _i32),
            pltpu.VMEM((BLK,), _i32),
            pltpu.VMEM((BLK + 80,), _i32),
            pltpu.VMEM((BLK + 80,), _i32),
            pltpu.VMEM((BLK + 80,), _f32),
            pltpu.VMEM((RW_DIM,), _f32),
            pltpu.VMEM((NODES_PER_TILE,), _f32),
            pltpu.VMEM((NODES_PER_TILE,), _f32),
            pltpu.VMEM((GATHER_CHUNK, D), _f32),
            pltpu.VMEM((NODES_PER_TILE, D), _f32),
            pltpu.SemaphoreType.DMA,
        ],
        compiler_params=pltpu.CompilerParams(needs_layout_passes=False),
    )
    return k(xw, dst, et, src, relw_l, inv_l, invc, zeros_a)


_ROWS = 256
_GRID = NPAD // _ROWS


def _mm2_kernel(x_ref, w_ref, wr_ref, xw_ref, xt_ref):
    h = x_ref[...]
    xw_ref[...] = jnp.dot(h, w_ref[...], preferred_element_type=_f32)
    xt_ref[...] = jnp.dot(h, wr_ref[...], preferred_element_type=_f32)


def _run_mm2(x, w, wr):
    return pl.pallas_call(
        _mm2_kernel,
        grid=(_GRID,),
        in_specs=[
            pl.BlockSpec((_ROWS, D), lambda i: (i, 0)),
            pl.BlockSpec((D, D), lambda i: (0, 0)),
            pl.BlockSpec((D, D), lambda i: (0, 0)),
        ],
        out_specs=[
            pl.BlockSpec((_ROWS, D), lambda i: (i, 0)),
            pl.BlockSpec((_ROWS, D), lambda i: (i, 0)),
        ],
        out_shape=[
            jax.ShapeDtypeStruct((NPAD, D), _f32),
            jax.ShapeDtypeStruct((NPAD, D), _f32),
        ],
    )(x, w, wr)


def _fuse_kernel(m_ref, xt_ref, b_ref, oh_ref, w_ref, wr_ref,
                 xw_ref, xtn_ref, pool_ref):
    i = pl.program_id(0)
    h = jnp.maximum(m_ref[...] + xt_ref[...] + b_ref[0:1, :], 0.0)
    xw_ref[...] = jnp.dot(h, w_ref[...], preferred_element_type=_f32)
    xtn_ref[...] = jnp.dot(h, wr_ref[...], preferred_element_type=_f32)
    acc = lax.dot_general(oh_ref[...], h, (((0,), (0,)), ((), ())),
                          preferred_element_type=_f32)

    @pl.when(i == 0)
    def _():
        pool_ref[...] = jnp.zeros_like(pool_ref)

    pool_ref[...] += acc


def _run_fuse(m, xt, b8, oh, w, wr):
    return pl.pallas_call(
        _fuse_kernel,
        grid=(_GRID,),
        in_specs=[
            pl.BlockSpec((_ROWS, D), lambda i: (i, 0)),
            pl.BlockSpec((_ROWS, D), lambda i: (i, 0)),
            pl.BlockSpec((8, D), lambda i: (0, 0)),
            pl.BlockSpec((_ROWS, 128), lambda i: (i, 0)),
            pl.BlockSpec((D, D), lambda i: (0, 0)),
            pl.BlockSpec((D, D), lambda i: (0, 0)),
        ],
        out_specs=[
            pl.BlockSpec((_ROWS, D), lambda i: (i, 0)),
            pl.BlockSpec((_ROWS, D), lambda i: (i, 0)),
            pl.BlockSpec((128, D), lambda i: (0, 0)),
        ],
        out_shape=[
            jax.ShapeDtypeStruct((NPAD, D), _f32),
            jax.ShapeDtypeStruct((NPAD, D), _f32),
            jax.ShapeDtypeStruct((128, D), _f32),
        ],
    )(m, xt, b8, oh, w, wr)


def _final_kernel(m_ref, xt_ref, b_ref, oh_ref, p0_ref, p1_ref,
                  mw1_ref, mb1_ref, mw2_ref, mb2_ref,
                  out_ref, pool_ref, cnt_ref):
    i = pl.program_id(0)
    h = jnp.maximum(m_ref[...] + xt_ref[...] + b_ref[0:1, :], 0.0)
    acc = lax.dot_general(oh_ref[...], h, (((0,), (0,)), ((), ())),
                          preferred_element_type=_f32)
    cacc = lax.dot_general(oh_ref[...], jnp.ones_like(h),
                           (((0,), (0,)), ((), ())),
                           preferred_element_type=_f32)

    @pl.when(i == 0)
    def _():
        pool_ref[...] = jnp.zeros_like(pool_ref)
        cnt_ref[...] = jnp.zeros_like(cnt_ref)

    pool_ref[...] += acc
    cnt_ref[...] += cacc

    @pl.when(i == _GRID - 1)
    def _():
        gr = (p0_ref[...] + p1_ref[...] + pool_ref[...]) / jnp.maximum(
            cnt_ref[...], 1.0)
        h1 = jnp.maximum(
            jnp.dot(gr, mw1_ref[...], preferred_element_type=_f32)
            + mb1_ref[0:1, :], 0.0)
        o = jnp.dot(h1, mw2_ref[...], preferred_element_type=_f32) \
            + mb2_ref[0:1, :]
        out_ref[...] = o[0:NG, :]


def _run_final(m, xt, b8, oh, p0, p1, mw1, mb18, mw2, mb28):
    return pl.pallas_call(
        _final_kernel,
        grid=(_GRID,),
        in_specs=[
            pl.BlockSpec((_ROWS, D), lambda i: (i, 0)),
            pl.BlockSpec((_ROWS, D), lambda i: (i, 0)),
            pl.BlockSpec((8, D), lambda i: (0, 0)),
            pl.BlockSpec((_ROWS, 128), lambda i: (i, 0)),
            pl.BlockSpec((128, D), lambda i: (0, 0)),
            pl.BlockSpec((128, D), lambda i: (0, 0)),
            pl.BlockSpec((D, D), lambda i: (0, 0)),
            pl.BlockSpec((8, D), lambda i: (0, 0)),
            pl.BlockSpec((D, 128), lambda i: (0, 0)),
            pl.BlockSpec((8, 128), lambda i: (0, 0)),
        ],
        out_specs=[
            pl.BlockSpec((NG, 128), lambda i: (0, 0)),
            pl.BlockSpec((128, D), lambda i: (0, 0)),
            pl.BlockSpec((128, D), lambda i: (0, 0)),
        ],
        out_shape=[
            jax.ShapeDtypeStruct((NG, 128), _f32),
            jax.ShapeDtypeStruct((128, D), _f32),
            jax.ShapeDtypeStruct((128, D), _f32),
        ],
    )(m, xt, b8, oh, p0, p1, mw1, mb18, mw2, mb28)


def kernel(x, edge_index, edge_attr, residue_type, batch,
           W0, Wr0, b0, rw0, W1, Wr1, b1, rw1, W2, Wr2, b2, rw2,
           mW1, mb1, mW2, mb2):
    loop = jnp.arange(N, dtype=_i32)
    src = jnp.concatenate([edge_index[0], loop])
    dst = jnp.concatenate([edge_index[1], loop])
    et = jnp.concatenate([edge_attr, residue_type + NUM_EDGE_TYPES])
    npad = EPAD - (E + N)
    src = jnp.concatenate([src, jnp.zeros((npad,), _i32)])
    dst = jnp.concatenate([dst, jnp.full((npad,), PAD_DST, _i32)])
    et = jnp.concatenate([et, jnp.full((npad,), 440, _i32)])

    def prep_rw(rw):
        r = jax.nn.leaky_relu(rw * 1.0)
        return jnp.concatenate([r, jnp.zeros((RW_DIM - r.shape[0],), _f32)])

    relw = jnp.stack([prep_rw(rw0), prep_rw(rw1), prep_rw(rw2)])

    xp = jnp.concatenate([x, jnp.zeros((NPAD - N, x.shape[1]), _f32)])
    oh = (batch[:, None] == jnp.arange(8, dtype=batch.dtype)[None, :])
    oh = oh.astype(_f32)
    oh = jnp.pad(oh, ((0, NPAD - N), (0, 120)))

    zeros4 = jnp.zeros((4, NPAD), _f32)
    zeros_a = jnp.zeros((NODES_PER_TILE, D), _f32)

    part = _run_deg(dst, et, relw, zeros4)
    norm = _run_reduce(part)
    inv0, inv1, inv2, invc = norm[0], norm[1], norm[2], norm[3]

    def b_tile(b):
        return jnp.broadcast_to(b[None, :], (8, b.shape[0]))

    xw0, xt0 = _run_mm2(xp, W0, Wr0)
    m0 = _run_msg(xw0, dst, et, src, relw[0], inv0, invc, zeros_a)
    xw1, xt1, p0 = _run_fuse(m0, xt0, b_tile(b0), oh, W1, Wr1)
    m1 = _run_msg(xw1, dst, et, src, relw[1], inv1, invc, zeros_a)
    xw2, xt2, p1 = _run_fuse(m1, xt1, b_tile(b1), oh, W2, Wr2)
    m2 = _run_msg(xw2, dst, et, src, relw[2], inv2, invc, zeros_a)
    out, _, _ = _run_final(m2, xt2, b_tile(b2), oh, p0, p1,
                           mW1, b_tile(mb1), mW2, b_tile(mb2))
    return out


# speedup vs baseline: 1.8768x; 1.0929x over previous
import functools

import jax
import jax.numpy as jnp
from jax import lax
from jax.experimental import pallas as pl
from jax.experimental.pallas import tpu as pltpu
from jax.experimental.pallas import tpu_sc as plsc

N = 10000
E = 160000
D = 256
NG = 8
NUM_EDGE_TYPES = 400
RW_DIM = 448
NPAD = 10240
NODES_PER_TILE = 320
NTILES = 32
BLK = 2048
EPAD = 172032
NBLK = EPAD // BLK
GATHER_CHUNK = 32
PAD_DST = 1 << 29

_f32 = jnp.float32
_i32 = jnp.int32


def _sc_mesh():
    return plsc.VectorSubcoreMesh(core_axis_name="c", subcore_axis_name="s")


def _wid():
    return lax.axis_index("s") * 2 + lax.axis_index("c")


def _deg_kernel(dst_hbm, et_hbm, relw_hbm, zeros_hbm, part_hbm,
                dst_v, et_v, relw_v, acc_v):
    wid = _wid()
    epw = EPAD // NTILES
    base = wid * epw
    pltpu.sync_copy(dst_hbm.at[pl.ds(base, epw)], dst_v)
    pltpu.sync_copy(et_hbm.at[pl.ds(base, epw)], et_v)
    pltpu.sync_copy(relw_hbm, relw_v)
    pltpu.sync_copy(zeros_hbm, acc_v)
    col = lax.iota(_i32, 16)
    ones16 = jnp.ones((16,), _f32)

    def body(v, _):
        d16 = plsc.load_gather(dst_v, [v * 16 + col])
        e16 = plsc.load_gather(et_v, [v * 16 + col])
        m = d16 < PAD_DST
        dc = jnp.where(m, d16, 0)
        for l in range(3):
            wl = plsc.load_gather(relw_v, [jnp.full((16,), l, _i32), e16])
            plsc.addupdate_scatter(acc_v, [jnp.full((16,), l, _i32), dc],
                                   wl, mask=m)
        plsc.addupdate_scatter(acc_v, [jnp.full((16,), 3, _i32), dc],
                               ones16, mask=m)
        return 0

    lax.fori_loop(0, epw // 16, body, 0)
    pltpu.sync_copy(acc_v, part_hbm.at[wid])


def _run_deg(dst, et, relw_all, zeros4):
    k = pl.kernel(
        _deg_kernel,
        out_type=jax.ShapeDtypeStruct((NTILES, 4, NPAD), _f32),
        mesh=_sc_mesh(),
        scratch_types=[
            pltpu.VMEM((EPAD // NTILES,), _i32),
            pltpu.VMEM((EPAD // NTILES,), _i32),
            pltpu.VMEM((3, RW_DIM), _f32),
            pltpu.VMEM((4, NPAD), _f32),
        ],
        compiler_params=pltpu.CompilerParams(needs_layout_passes=False),
    )
    return k(dst, et, relw_all, zeros4)


def _reduce_kernel(part_ref, out_ref):
    d = jnp.sum(part_ref[...], axis=0)
    row = lax.broadcasted_iota(_i32, d.shape, 0)
    safe = jnp.maximum(d, 1.0)
    out_ref[...] = jnp.where(row < 3, 1.0 / d, 1.0 / safe)


def _run_reduce(part):
    return pl.pallas_call(
        _reduce_kernel,
        out_shape=jax.ShapeDtypeStruct((4, NPAD), _f32),
    )(part)


def _msg_kernel(xw_hbm, dst_hbm, et_hbm, src_hbm, relw_hbm, inv_hbm,
                invc_hbm, zeros_hbm, out_hbm,
                s_dst0, s_et0, s_src0, s_dst1, s_et1, s_src1,
                c_src, c_dstl, c_ew,
                relw_v, inv_v, invc_v, rows0, rows1, agg_v,
                sem_i0, sem_i1, sem_r0, sem_r1):
    wid = _wid()
    lo = wid * NODES_PER_TILE
    pltpu.sync_copy(relw_hbm, relw_v)
    pltpu.sync_copy(inv_hbm.at[pl.ds(lo, NODES_PER_TILE)], inv_v)
    pltpu.sync_copy(invc_hbm.at[pl.ds(lo, NODES_PER_TILE)], invc_v)
    pltpu.sync_copy(zeros_hbm, agg_v)
    col = lax.iota(_i32, 16)
    zf16 = jnp.zeros((16,), _f32)
    zi16 = jnp.zeros((16,), _i32)
    idx_bufs = ((s_dst0, s_et0, s_src0, sem_i0),
                (s_dst1, s_et1, s_src1, sem_i1))
    row_bufs = ((rows0, sem_r0), (rows1, sem_r1))

    def fire_idx(b, which):
        sd, se, ss, sem = idx_bufs[which]
        off = b * BLK
        pltpu.async_copy(dst_hbm.at[pl.ds(off, BLK)], sd, sem)
        pltpu.async_copy(et_hbm.at[pl.ds(off, BLK)], se, sem)
        pltpu.async_copy(src_hbm.at[pl.ds(off, BLK)], ss, sem)

    def wait_idx(which):
        sd, se, ss, sem = idx_bufs[which]
        pltpu.make_async_copy(dst_hbm.at[pl.ds(0, BLK)], sd, sem).wait()
        pltpu.make_async_copy(et_hbm.at[pl.ds(0, BLK)], se, sem).wait()
        pltpu.make_async_copy(src_hbm.at[pl.ds(0, BLK)], ss, sem).wait()

    def fire_rows(g, which):
        buf, sem = row_bufs[which]
        pltpu.async_copy(
            xw_hbm.at[c_src.at[pl.ds(g * GATHER_CHUNK, GATHER_CHUNK)]],
            buf, sem)

    def wait_rows(which):
        buf, sem = row_bufs[which]
        pltpu.make_async_copy(
            xw_hbm.at[c_src.at[pl.ds(0, GATHER_CHUNK)]], buf, sem).wait()

    def acc_rows(g, which):
        buf, _ = row_bufs[which]

        def rbody(k, _c):
            idxk = jnp.full((16,), g * GATHER_CHUNK + k, _i32)
            ewk = plsc.load_gather(c_ew, [idxk])
            dk = plsc.load_gather(c_dstl, [idxk])
            kf = jnp.full((16,), k, _i32)
            for j in range(16):
                val = plsc.load_gather(buf, [kf, j * 16 + col]) * ewk
                plsc.addupdate_scatter(agg_v, [dk, j * 16 + col], val)
            return 0

        lax.fori_loop(0, GATHER_CHUNK, rbody, 0)

    def process_block(which):
        sd, se, ss, _ = idx_bufs[which]

        def cbody(v, cursor):
            d16 = plsc.load_gather(sd, [v * 16 + col])
            m = (d16 >= lo) & (d16 < lo + NODES_PER_TILE)
            e16 = plsc.load_gather(se, [v * 16 + col])
            s16 = plsc.load_gather(ss, [v * 16 + col])
            w16 = plsc.load_gather(relw_v, [e16])
            dl = jnp.where(m, d16 - lo, 0)
            iv = plsc.load_gather(inv_v, [dl])
            ew = w16 * iv
            plsc.store_compressed(c_src.at[pl.ds(cursor, 16)], s16, mask=m)
            plsc.store_compressed(c_dstl.at[pl.ds(cursor, 16)], dl, mask=m)
            plsc.store_compressed(c_ew.at[pl.ds(cursor, 16)], ew, mask=m)
            return cursor + jnp.sum(m.astype(_i32))

        kcnt = lax.fori_loop(0, BLK // 16, cbody, jnp.int32(0))
        for t in range(4):
            plsc.store_scatter(c_ew, [kcnt + t * 16 + col], zf16)
            plsc.store_scatter(c_src, [kcnt + t * 16 + col], zi16)
            plsc.store_scatter(c_dstl, [kcnt + t * 16 + col], zi16)
        nch = (kcnt + GATHER_CHUNK - 1) // GATHER_CHUNK

        @pl.when(nch > 0)
        def _():
            fire_rows(0, 0)

            def pair_body(h, _c):
                g0 = 2 * h

                @pl.when(g0 + 1 < nch)
                def _():
                    fire_rows(g0 + 1, 1)

                wait_rows(0)
                acc_rows(g0, 0)

                @pl.when(g0 + 1 < nch)
                def _():
                    @pl.when(g0 + 2 < nch)
                    def _():
                        fire_rows(g0 + 2, 0)

                    wait_rows(1)
                    acc_rows(g0 + 1, 1)

                return 0

            lax.fori_loop(0, (nch + 1) // 2, pair_body, 0)

    fire_idx(0, 0)

    def block_pair(h, _c):
        b0 = 2 * h
        fire_idx(b0 + 1, 1)
        wait_idx(0)
        process_block(0)

        @pl.when(b0 + 2 < NBLK)
        def _():
            fire_idx(b0 + 2, 0)

        wait_idx(1)
        process_block(1)
        return 0

    lax.fori_loop(0, NBLK // 2, block_pair, 0)

    def fbody(r, _):
        rf = jnp.full((16,), r, _i32)
        ic = plsc.load_gather(invc_v, [rf])
        for j in range(16):
            v = plsc.load_gather(agg_v, [rf, j * 16 + col]) * ic
            plsc.store_scatter(agg_v, [rf, j * 16 + col], v)
        return 0

    lax.fori_loop(0, NODES_PER_TILE, fbody, 0)
    pltpu.sync_copy(agg_v, out_hbm.at[pl.ds(lo, NODES_PER_TILE)])


def _run_msg(xw, dst, et, src, relw_l, inv_l, invc, zeros_a):
    k = pl.kernel(
        _msg_kernel,
        out_type=jax.ShapeDtypeStruct((NPAD, D), _f32),
        mesh=_sc_mesh(),
        scratch_types=[
            pltpu.VMEM((BLK,), _i32),
            pltpu.VMEM((BLK,), _i32),
            pltpu.VMEM((BLK,), _i32),
            pltpu.VMEM((BLK,), _i32),
            pltpu.VMEM((BLK,), _i32),
            pltpu.VMEM((BLK,), _i32),
            pltpu.VMEM((BLK + 80,), _i32),
            pltpu.VMEM((BLK + 80,), _i32),
            pltpu.VMEM((BLK + 80,), _f32),
            pltpu.VMEM((RW_DIM,), _f32),
            pltpu.VMEM((NODES_PER_TILE,), _f32),
            pltpu.VMEM((NODES_PER_TILE,), _f32),
            pltpu.VMEM((GATHER_CHUNK, D), _f32),
            pltpu.VMEM((GATHER_CHUNK, D), _f32),
            pltpu.VMEM((NODES_PER_TILE, D), _f32),
            pltpu.SemaphoreType.DMA,
            pltpu.SemaphoreType.DMA,
            pltpu.SemaphoreType.DMA,
            pltpu.SemaphoreType.DMA,
        ],
        compiler_params=pltpu.CompilerParams(needs_layout_passes=False),
    )
    return k(xw, dst, et, src, relw_l, inv_l, invc, zeros_a)


_ROWS = 256
_GRID = NPAD // _ROWS


def _mm2_kernel(x_ref, w_ref, wr_ref, xw_ref, xt_ref):
    h = x_ref[...]
    xw_ref[...] = jnp.dot(h, w_ref[...], preferred_element_type=_f32)
    xt_ref[...] = jnp.dot(h, wr_ref[...], preferred_element_type=_f32)


def _run_mm2(x, w, wr):
    return pl.pallas_call(
        _mm2_kernel,
        grid=(_GRID,),
        in_specs=[
            pl.BlockSpec((_ROWS, D), lambda i: (i, 0)),
            pl.BlockSpec((D, D), lambda i: (0, 0)),
            pl.BlockSpec((D, D), lambda i: (0, 0)),
        ],
        out_specs=[
            pl.BlockSpec((_ROWS, D), lambda i: (i, 0)),
            pl.BlockSpec((_ROWS, D), lambda i: (i, 0)),
        ],
        out_shape=[
            jax.ShapeDtypeStruct((NPAD, D), _f32),
            jax.ShapeDtypeStruct((NPAD, D), _f32),
        ],
    )(x, w, wr)


def _fuse_kernel(m_ref, xt_ref, b_ref, oh_ref, w_ref, wr_ref,
                 xw_ref, xtn_ref, pool_ref):
    i = pl.program_id(0)
    h = jnp.maximum(m_ref[...] + xt_ref[...] + b_ref[0:1, :], 0.0)
    xw_ref[...] = jnp.dot(h, w_ref[...], preferred_element_type=_f32)
    xtn_ref[...] = jnp.dot(h, wr_ref[...], preferred_element_type=_f32)
    acc = lax.dot_general(oh_ref[...], h, (((0,), (0,)), ((), ())),
                          preferred_element_type=_f32)

    @pl.when(i == 0)
    def _():
        pool_ref[...] = jnp.zeros_like(pool_ref)

    pool_ref[...] += acc


def _run_fuse(m, xt, b8, oh, w, wr):
    return pl.pallas_call(
        _fuse_kernel,
        grid=(_GRID,),
        in_specs=[
            pl.BlockSpec((_ROWS, D), lambda i: (i, 0)),
            pl.BlockSpec((_ROWS, D), lambda i: (i, 0)),
            pl.BlockSpec((8, D), lambda i: (0, 0)),
            pl.BlockSpec((_ROWS, 128), lambda i: (i, 0)),
            pl.BlockSpec((D, D), lambda i: (0, 0)),
            pl.BlockSpec((D, D), lambda i: (0, 0)),
        ],
        out_specs=[
            pl.BlockSpec((_ROWS, D), lambda i: (i, 0)),
            pl.BlockSpec((_ROWS, D), lambda i: (i, 0)),
            pl.BlockSpec((128, D), lambda i: (0, 0)),
        ],
        out_shape=[
            jax.ShapeDtypeStruct((NPAD, D), _f32),
            jax.ShapeDtypeStruct((NPAD, D), _f32),
            jax.ShapeDtypeStruct((128, D), _f32),
        ],
    )(m, xt, b8, oh, w, wr)


def _final_kernel(m_ref, xt_ref, b_ref, oh_ref, p0_ref, p1_ref,
                  mw1_ref, mb1_ref, mw2_ref, mb2_ref,
                  out_ref, pool_ref, cnt_ref):
    i = pl.program_id(0)
    h = jnp.maximum(m_ref[...] + xt_ref[...] + b_ref[0:1, :], 0.0)
    acc = lax.dot_general(oh_ref[...], h, (((0,), (0,)), ((), ())),
                          preferred_element_type=_f32)
    cacc = lax.dot_general(oh_ref[...], jnp.ones_like(h),
                           (((0,), (0,)), ((), ())),
                           preferred_element_type=_f32)

    @pl.when(i == 0)
    def _():
        pool_ref[...] = jnp.zeros_like(pool_ref)
        cnt_ref[...] = jnp.zeros_like(cnt_ref)

    pool_ref[...] += acc
    cnt_ref[...] += cacc

    @pl.when(i == _GRID - 1)
    def _():
        gr = (p0_ref[...] + p1_ref[...] + pool_ref[...]) / jnp.maximum(
            cnt_ref[...], 1.0)
        h1 = jnp.maximum(
            jnp.dot(gr, mw1_ref[...], preferred_element_type=_f32)
            + mb1_ref[0:1, :], 0.0)
        o = jnp.dot(h1, mw2_ref[...], preferred_element_type=_f32) \
            + mb2_ref[0:1, :]
        out_ref[...] = o[0:NG, :]


def _run_final(m, xt, b8, oh, p0, p1, mw1, mb18, mw2, mb28):
    return pl.pallas_call(
        _final_kernel,
        grid=(_GRID,),
        in_specs=[
            pl.BlockSpec((_ROWS, D), lambda i: (i, 0)),
            pl.BlockSpec((_ROWS, D), lambda i: (i, 0)),
            pl.BlockSpec((8, D), lambda i: (0, 0)),
            pl.BlockSpec((_ROWS, 128), lambda i: (i, 0)),
            pl.BlockSpec((128, D), lambda i: (0, 0)),
            pl.BlockSpec((128, D), lambda i: (0, 0)),
            pl.BlockSpec((D, D), lambda i: (0, 0)),
            pl.BlockSpec((8, D), lambda i: (0, 0)),
            pl.BlockSpec((D, 128), lambda i: (0, 0)),
            pl.BlockSpec((8, 128), lambda i: (0, 0)),
        ],
        out_specs=[
            pl.BlockSpec((NG, 128), lambda i: (0, 0)),
            pl.BlockSpec((128, D), lambda i: (0, 0)),
            pl.BlockSpec((128, D), lambda i: (0, 0)),
        ],
        out_shape=[
            jax.ShapeDtypeStruct((NG, 128), _f32),
            jax.ShapeDtypeStruct((128, D), _f32),
            jax.ShapeDtypeStruct((128, D), _f32),
        ],
    )(m, xt, b8, oh, p0, p1, mw1, mb18, mw2, mb28)


def kernel(x, edge_index, edge_attr, residue_type, batch,
           W0, Wr0, b0, rw0, W1, Wr1, b1, rw1, W2, Wr2, b2, rw2,
           mW1, mb1, mW2, mb2):
    loop = jnp.arange(N, dtype=_i32)
    src = jnp.concatenate([edge_index[0], loop])
    dst = jnp.concatenate([edge_index[1], loop])
    et = jnp.concatenate([edge_attr, residue_type + NUM_EDGE_TYPES])
    npad = EPAD - (E + N)
    src = jnp.concatenate([src, jnp.zeros((npad,), _i32)])
    dst = jnp.concatenate([dst, jnp.full((npad,), PAD_DST, _i32)])
    et = jnp.concatenate([et, jnp.full((npad,), 440, _i32)])

    def prep_rw(rw):
        r = jax.nn.leaky_relu(rw * 1.0)
        return jnp.concatenate([r, jnp.zeros((RW_DIM - r.shape[0],), _f32)])

    relw = jnp.stack([prep_rw(rw0), prep_rw(rw1), prep_rw(rw2)])

    xp = jnp.concatenate([x, jnp.zeros((NPAD - N, x.shape[1]), _f32)])
    oh = (batch[:, None] == jnp.arange(8, dtype=batch.dtype)[None, :])
    oh = oh.astype(_f32)
    oh = jnp.pad(oh, ((0, NPAD - N), (0, 120)))

    zeros4 = jnp.zeros((4, NPAD), _f32)
    zeros_a = jnp.zeros((NODES_PER_TILE, D), _f32)

    part = _run_deg(dst, et, relw, zeros4)
    norm = _run_reduce(part)
    inv0, inv1, inv2, invc = norm[0], norm[1], norm[2], norm[3]

    def b_tile(b):
        return jnp.broadcast_to(b[None, :], (8, b.shape[0]))

    xw0, xt0 = _run_mm2(xp, W0, Wr0)
    m0 = _run_msg(xw0, dst, et, src, relw[0], inv0, invc, zeros_a)
    xw1, xt1, p0 = _run_fuse(m0, xt0, b_tile(b0), oh, W1, Wr1)
    m1 = _run_msg(xw1, dst, et, src, relw[1], inv1, invc, zeros_a)
    xw2, xt2, p1 = _run_fuse(m1, xt1, b_tile(b1), oh, W2, Wr2)
    m2 = _run_msg(xw2, dst, et, src, relw[2], inv2, invc, zeros_a)
    out, _, _ = _run_final(m2, xt2, b_tile(b2), oh, p0, p1,
                           mW1, b_tile(mb1), mW2, b_tile(mb2))
    return out
